# baseline (speedup 1.0000x reference)
import jax
import jax.numpy as jnp
from jax.experimental import pallas as pl

_NUM_COLS = 2048
_IN = 16384
_K = 40
_ROW_BLK = 256
_RANK_BLK = 256


def _matvec_body(inp_ref, conn_ref, out_ref):
    out_ref[:] = jnp.sum(conn_ref[:] * inp_ref[:], axis=1, keepdims=True)


def _rank_body(o_col_ref, o_row_ref, out_ref):
    step = pl.program_id(0)
    oc = o_col_ref[:]
    orow = o_row_ref[:]
    gt = (orow > oc).astype(jnp.float32)
    j_idx = jax.lax.broadcasted_iota(jnp.int32, (_RANK_BLK, _NUM_COLS), 1)
    i_idx = jax.lax.broadcasted_iota(jnp.int32, (_RANK_BLK, _NUM_COLS), 0)
    i_idx = i_idx + step * _RANK_BLK
    eq_before = jnp.where((orow == oc) & (j_idx < i_idx), 1.0, 0.0)
    rank = jnp.sum(gt + eq_before, axis=1, keepdims=True)
    out_ref[:] = (rank < float(_K)).astype(jnp.float32)


def kernel(input_vector, connections):
    inp2 = input_vector.reshape(1, _IN)
    overlap = pl.pallas_call(
        _matvec_body,
        grid=(_NUM_COLS // _ROW_BLK,),
        in_specs=[
            pl.BlockSpec((1, _IN), lambda i: (0, 0)),
            pl.BlockSpec((_ROW_BLK, _IN), lambda i: (i, 0)),
        ],
        out_specs=pl.BlockSpec((_ROW_BLK, 1), lambda i: (i, 0)),
        out_shape=jax.ShapeDtypeStruct((_NUM_COLS, 1), jnp.float32),
    )(inp2, connections)

    mask = pl.pallas_call(
        _rank_body,
        grid=(_NUM_COLS // _RANK_BLK,),
        in_specs=[
            pl.BlockSpec((_RANK_BLK, 1), lambda i: (i, 0)),
            pl.BlockSpec((1, _NUM_COLS), lambda i: (0, 0)),
        ],
        out_specs=pl.BlockSpec((_RANK_BLK, 1), lambda i: (i, 0)),
        out_shape=jax.ShapeDtypeStruct((_NUM_COLS, 1), jnp.float32),
    )(overlap, overlap.reshape(1, _NUM_COLS))
    return mask.reshape(_NUM_COLS)


# speedup vs baseline: 1.1513x; 1.1513x over previous
import jax
import jax.numpy as jnp
from jax.experimental import pallas as pl
from jax.experimental.pallas import tpu as pltpu

_N = 2048
_IN = 16384
_K = 40
_BLK = 128
_NB = _N // _BLK


def _fused_body(inp_ref, conn_ref, out_ref, ov_col, ov_row):
    i = pl.program_id(0)
    ov = jnp.sum(conn_ref[:] * inp_ref[:], axis=1)
    ov_row[:, pl.ds(i * _BLK, _BLK)] = ov.reshape(1, _BLK)
    ov_col[pl.ds(i * _BLK, _BLK), :] = ov.reshape(_BLK, 1)

    @pl.when(i == _NB - 1)
    def _rank_and_mask():
        orow = ov_row[:]
        tri = (
            jax.lax.broadcasted_iota(jnp.int32, (_BLK, _BLK), 1)
            < jax.lax.broadcasted_iota(jnp.int32, (_BLK, _BLK), 0)
        )
        for b in range(_NB):
            oc = ov_col[b * _BLK:(b + 1) * _BLK, :]
            gt = (orow > oc).astype(jnp.float32)
            rank = jnp.sum(gt, axis=1, keepdims=True)
            if b > 0:
                eq_lo = (orow[:, : b * _BLK] == oc).astype(jnp.float32)
                rank = rank + jnp.sum(eq_lo, axis=1, keepdims=True)
            eq_dg = jnp.where(
                (orow[:, b * _BLK:(b + 1) * _BLK] == oc) & tri, 1.0, 0.0
            )
            rank = rank + jnp.sum(eq_dg, axis=1, keepdims=True)
            out_ref[b * _BLK:(b + 1) * _BLK, :] = (
                rank < float(_K)
            ).astype(jnp.float32)


def kernel(input_vector, connections):
    mask = pl.pallas_call(
        _fused_body,
        grid=(_NB,),
        in_specs=[
            pl.BlockSpec((1, _IN), lambda i: (0, 0)),
            pl.BlockSpec((_BLK, _IN), lambda i: (i, 0)),
        ],
        out_specs=pl.BlockSpec((_N, 1), lambda i: (0, 0)),
        out_shape=jax.ShapeDtypeStruct((_N, 1), jnp.float32),
        scratch_shapes=[
            pltpu.VMEM((_N, 1), jnp.float32),
            pltpu.VMEM((1, _N), jnp.float32),
        ],
    )(input_vector.reshape(1, _IN), connections)
    return mask.reshape(_N)
